# baseline (speedup 1.0000x reference)
import jax
import jax.numpy as jnp
from jax.experimental import pallas as pl


_BK = 256


def _gate_body(x_ref, w_ref, o_ref):
    @pl.when(pl.program_id(0) == 0)
    def _():
        o_ref[...] = jnp.zeros_like(o_ref)

    o_ref[...] += jax.lax.dot_general(
        x_ref[...],
        w_ref[...],
        dimension_numbers=(((1,), (1,)), ((), ())),
        preferred_element_type=jnp.float32,
    )


def kernel(x, gate_weight):
    M, K = x.shape
    E = gate_weight.shape[0]
    return pl.pallas_call(
        _gate_body,
        grid=(K // _BK,),
        in_specs=[
            pl.BlockSpec((M, _BK), lambda k: (0, k)),
            pl.BlockSpec((E, _BK), lambda k: (0, k)),
        ],
        out_specs=pl.BlockSpec((M, E), lambda k: (0, 0)),
        out_shape=jax.ShapeDtypeStruct((M, E), jnp.float32),
    )(x, gate_weight)


# speedup vs baseline: 1.2272x; 1.2272x over previous
import jax
import jax.numpy as jnp
from jax.experimental import pallas as pl

_BM = 1024


def _gate_body(x_ref, w_ref, o_ref):
    o_ref[...] = jax.lax.dot_general(
        w_ref[...],
        x_ref[...],
        dimension_numbers=(((1,), (1,)), ((), ())),
        preferred_element_type=jnp.float32,
    )


def kernel(x, gate_weight):
    M, K = x.shape
    E = gate_weight.shape[0]
    out_t = pl.pallas_call(
        _gate_body,
        grid=(M // _BM,),
        in_specs=[
            pl.BlockSpec((_BM, K), lambda i: (i, 0)),
            pl.BlockSpec((E, K), lambda i: (0, 0)),
        ],
        out_specs=pl.BlockSpec((E, _BM), lambda i: (0, i)),
        out_shape=jax.ShapeDtypeStruct((E, M), jnp.float32),
    )(x, gate_weight)
    return out_t.T
